# baseline (speedup 1.0000x reference)
import functools

import jax
import jax.numpy as jnp
from jax import lax
from jax.experimental import pallas as pl
from jax.experimental.pallas import tpu as pltpu
from jax.experimental.pallas import tpu_sc as plsc

_W, _H, _D = 1024, 1024, 64
_N = 524288
_NC = 2
_NS = 16
_NW = _NC * _NS
_L = 16
_CELLS = _W * _H

_RB = 256
_CELLW = _CELLS // _NW
_NBCH = _CELLW // _RB

_C = 128
_PW = _N // _NW
_NCH = _PW // _C
_G = _C // _L

_params = pltpu.CompilerParams(
    needs_layout_passes=False, use_tc_tiling_on_sc=True)
_mesh = plsc.VectorSubcoreMesh(core_axis_name="c", subcore_axis_name="s")


def _bp_src(chunk, wid, ptr):
    c0 = wid * _CELLW + chunk * _RB
    xr = pl.multiple_of(lax.shift_right_logical(c0, 10) * _D, _D)
    yc = pl.multiple_of(c0 & (_H - 1), _RB)
    return ptr.at[pl.ds(xr, _D), pl.ds(yc, _RB)], c0


def _bp_prepare(chunk, wid, ptr, stage, sem):
    src, _ = _bp_src(chunk, wid, ptr)
    pltpu.async_copy(src, stage, sem)


def _bp_compute(chunk, wid, ptr, dup_hbm, stage, ob, ob_prev, sem):
    src, c0 = _bp_src(chunk, wid, ptr)
    pltpu.make_async_copy(src, stage, sem).wait()

    lane = lax.iota(jnp.int32, _L)
    zeros = jnp.zeros((_L,), jnp.int32)

    @pl.when(chunk > 0)
    def _():
        for k in range(_D // _L):
            ob_prev[_RB - 1, pl.ds(_D + k * _L, _L)] = plsc.load_gather(
                stage, [lane + k * _L, zeros])
        pltpu.sync_copy(ob_prev, dup_hbm.at[pl.ds(c0 - _RB, _RB)])

    @plsc.parallel_loop(0, _RB - 1, unroll=2)
    def body(r):
        rv = zeros + r
        for k in range(_D // _L):
            dl = lane + k * _L
            ob[r, pl.ds(k * _L, _L)] = plsc.load_gather(stage, [dl, rv])
            ob[r, pl.ds(_D + k * _L, _L)] = plsc.load_gather(
                stage, [dl, rv + 1])

    for k in range(_D // _L):
        ob[_RB - 1, pl.ds(k * _L, _L)] = plsc.load_gather(
            stage, [lane + k * _L, zeros + (_RB - 1)])


@functools.partial(
    pl.kernel,
    out_type=jax.ShapeDtypeStruct((_CELLS, 2 * _D), jnp.float32),
    mesh=_mesh,
    compiler_params=_params,
    scratch_types=[
        pltpu.VMEM((_D, _RB), jnp.float32),
        pltpu.VMEM((_RB, 2 * _D), jnp.float32),
        pltpu.SemaphoreType.DMA,
        pltpu.VMEM((_D, _RB), jnp.float32),
        pltpu.VMEM((_RB, 2 * _D), jnp.float32),
        pltpu.SemaphoreType.DMA,
    ],
)
def _build_pairs(plane_t, dup_hbm,
                 stageA, obA, semA, stageB, obB, semB):
    ptr = plane_t.reshape(_W * _D, _H)
    wid = lax.axis_index("s") * _NC + lax.axis_index("c")
    _bp_prepare(0, wid, ptr, stageA, semA)

    def pair(p, carry):
        g = p * 2
        _bp_prepare(g + 1, wid, ptr, stageB, semB)
        _bp_compute(g, wid, ptr, dup_hbm, stageA, obA, obB, semA)

        @pl.when(g + 2 < _NBCH)
        def _():
            _bp_prepare(g + 2, wid, ptr, stageA, semA)

        _bp_compute(g + 1, wid, ptr, dup_hbm, stageB, obB, obA, semB)
        return carry

    lax.fori_loop(0, _NBCH // 2, pair, 0)
    base = wid * _CELLW + (_NBCH - 1) * _RB
    pltpu.sync_copy(obB, dup_hbm.at[pl.ds(base, _RB)])


def _prepare(chunk, wid, xt_hbm, dup_hbm, xv, yv, idxb, wb, rows, gsem):
    base = wid * _PW + chunk * _C
    pltpu.sync_copy(xt_hbm.at[0, pl.ds(base, _C)], xv)
    pltpu.sync_copy(xt_hbm.at[1, pl.ds(base, _C)], yv)
    for g in range(_G):
        sl = pl.ds(g * _L, _L)
        xs = xv[sl]
        ys = yv[sl]
        x0 = xs.astype(jnp.int32)
        y0 = ys.astype(jnp.int32)
        tx = xs - x0.astype(jnp.float32)
        ty = ys - y0.astype(jnp.float32)
        c00 = x0 * _H + y0
        idxb[pl.ds(0 * _C + g * _L, _L)] = c00
        idxb[pl.ds(1 * _C + g * _L, _L)] = c00 + _H
        wb[pl.ds(0 * _C + g * _L, _L)] = tx
        wb[pl.ds(1 * _C + g * _L, _L)] = ty
    for c in range(2):
        pltpu.async_copy(dup_hbm.at[idxb.at[pl.ds(c * _C, _C)]], rows.at[c],
                         gsem)


def _compute(half, dup_hbm, idxb, wb, rows, ob, gsem):
    for c in range(2):
        pltpu.make_async_copy(dup_hbm.at[idxb.at[pl.ds(c * _C, _C)]],
                              rows.at[c], gsem).wait()

    lane = lax.iota(jnp.int32, _L)
    zeros = jnp.zeros((_L,), jnp.int32)

    @plsc.parallel_loop(0, _C, unroll=2)
    def body(i):
        iv = zeros + i
        txv = plsc.load_gather(wb, [iv])
        tyv = plsc.load_gather(wb, [iv + _C])
        col = zeros + (i + half * _C)
        for k in range(_D // _L):
            lo = pl.ds(k * _L, _L)
            hi = pl.ds(_D + k * _L, _L)
            p00 = rows[0, i, lo]
            p01 = rows[0, i, hi]
            p10 = rows[1, i, lo]
            p11 = rows[1, i, hi]
            top = p00 + txv * (p10 - p00)
            bot = p01 + txv * (p11 - p01)
            res = top + tyv * (bot - top)
            plsc.store_scatter(ob, [lane + k * _L, col], res)


@functools.partial(
    pl.kernel,
    out_type=jax.ShapeDtypeStruct((_D, _N), jnp.float32),
    mesh=_mesh,
    compiler_params=_params,
    scratch_types=[
        pltpu.VMEM((_C,), jnp.float32),
        pltpu.VMEM((_C,), jnp.float32),
        pltpu.VMEM((2 * _C,), jnp.int32),
        pltpu.VMEM((2 * _C,), jnp.float32),
        pltpu.VMEM((2, _C, 2 * _D), jnp.float32),
        pltpu.SemaphoreType.DMA,
        pltpu.VMEM((_C,), jnp.float32),
        pltpu.VMEM((_C,), jnp.float32),
        pltpu.VMEM((2 * _C,), jnp.int32),
        pltpu.VMEM((2 * _C,), jnp.float32),
        pltpu.VMEM((2, _C, 2 * _D), jnp.float32),
        pltpu.SemaphoreType.DMA,
        pltpu.VMEM((2, _D, 2 * _C), jnp.float32),
        pltpu.SemaphoreType.DMA,
        pltpu.SemaphoreType.DMA,
    ],
)
def _bilerp_sc(xt_hbm, dup_hbm, out_hbm,
               xvA, yvA, idxA, wbA, rowsA, gsemA,
               xvB, yvB, idxB, wbB, rowsB, gsemB, ob, osem0, osem1):
    wid = lax.axis_index("s") * _NC + lax.axis_index("c")
    slotA = (xvA, yvA, idxA, wbA, rowsA, gsemA)
    slotB = (xvB, yvB, idxB, wbB, rowsB, gsemB)
    _prepare(0, wid, xt_hbm, dup_hbm, *slotA)

    def quad(q, carry):
        for s, osem in ((0, osem0), (1, osem1)):
            g = q * 4 + s * 2
            obs = ob.at[s]
            colbase = wid * _PW + g * _C
            dst = out_hbm.at[:, pl.ds(colbase, 2 * _C)]
            _prepare(g + 1, wid, xt_hbm, dup_hbm, *slotB)

            @pl.when(q > 0)
            def _():
                pltpu.make_async_copy(obs, dst, osem).wait()

            _compute(0, dup_hbm, idxA, wbA, rowsA, obs, gsemA)

            @pl.when(g + 2 < _NCH)
            def _():
                _prepare(g + 2, wid, xt_hbm, dup_hbm, *slotA)

            _compute(1, dup_hbm, idxB, wbB, rowsB, obs, gsemB)
            pltpu.async_copy(obs, dst, osem)
        return carry

    lax.fori_loop(0, _NCH // 4, quad, 0)
    final = out_hbm.at[:, pl.ds(wid * _PW, 2 * _C)]
    pltpu.make_async_copy(ob.at[0], final, osem0).wait()
    pltpu.make_async_copy(ob.at[1], final, osem1).wait()


def kernel(x, plane):
    dup = _build_pairs(plane.transpose(0, 2, 1))
    out_t = _bilerp_sc(x.T, dup)
    return out_t.T


# speedup vs baseline: 1.7939x; 1.7939x over previous
import functools

import jax
import jax.numpy as jnp
from jax import lax
from jax.experimental import pallas as pl
from jax.experimental.pallas import tpu as pltpu
from jax.experimental.pallas import tpu_sc as plsc

_W, _H, _D = 1024, 1024, 64
_N = 524288
_NC = 2
_NS = 16
_NW = _NC * _NS
_L = 16
_CELLS = _W * _H

_RB = 128
_CELLW = _CELLS // _NW
_NBCH = _CELLW // _RB

_C = 128
_PW = _N // _NW
_NCH = _PW // _C
_G = _C // _L

_params = pltpu.CompilerParams(
    needs_layout_passes=False, use_tc_tiling_on_sc=True)
_mesh = plsc.VectorSubcoreMesh(core_axis_name="c", subcore_axis_name="s")


def _bp_prepare(chunk, wid, p64, stage, sem):
    base = wid * _CELLW + chunk * _RB
    pltpu.async_copy(p64.at[pl.ds(base, _RB)], stage.at[pl.ds(0, _RB)], sem)

    @pl.when(base + _RB < _CELLS)
    def _():
        pltpu.async_copy(p64.at[pl.ds(base + _RB, 8)],
                         stage.at[pl.ds(_RB, 8)], sem)


def _bp_compute(chunk, wid, p64, dup_hbm, stage, ob, sem):
    base = wid * _CELLW + chunk * _RB
    pltpu.make_async_copy(p64.at[pl.ds(base, _RB)], stage.at[pl.ds(0, _RB)],
                          sem).wait()

    @pl.when(base + _RB < _CELLS)
    def _():
        pltpu.make_async_copy(p64.at[pl.ds(base + _RB, 8)],
                              stage.at[pl.ds(_RB, 8)], sem).wait()

    @plsc.parallel_loop(0, _RB, unroll=2)
    def body(r):
        for k in range(_D // _L):
            sl = pl.ds(k * _L, _L)
            ob[r, sl] = stage[r, sl]
        for k in range(_D // _L):
            sl = pl.ds(k * _L, _L)
            ob[r, pl.ds(_D + k * _L, _L)] = stage[r + 1, sl]

    pltpu.sync_copy(ob, dup_hbm.at[pl.ds(base, _RB)])


@functools.partial(
    pl.kernel,
    out_type=jax.ShapeDtypeStruct((_CELLS, 2 * _D), jnp.float32),
    mesh=_mesh,
    compiler_params=_params,
    scratch_types=[
        pltpu.VMEM((_RB + 8, _D), jnp.float32),
        pltpu.VMEM((_RB, 2 * _D), jnp.float32),
        pltpu.SemaphoreType.DMA,
        pltpu.VMEM((_RB + 8, _D), jnp.float32),
        pltpu.VMEM((_RB, 2 * _D), jnp.float32),
        pltpu.SemaphoreType.DMA,
    ],
)
def _build_pairs(plane3d, dup_hbm,
                 stageA, obA, semA, stageB, obB, semB):
    p64 = plane3d.reshape(_CELLS, _D)
    wid = lax.axis_index("s") * _NC + lax.axis_index("c")
    _bp_prepare(0, wid, p64, stageA, semA)

    def pair(p, carry):
        g = p * 2
        _bp_prepare(g + 1, wid, p64, stageB, semB)
        _bp_compute(g, wid, p64, dup_hbm, stageA, obA, semA)

        @pl.when(g + 2 < _NBCH)
        def _():
            _bp_prepare(g + 2, wid, p64, stageA, semA)

        _bp_compute(g + 1, wid, p64, dup_hbm, stageB, obB, semB)
        return carry

    lax.fori_loop(0, _NBCH // 2, pair, 0)


def _prepare(chunk, wid, xt_hbm, dup_hbm, xv, yv, idxb, wb, rows, gsem):
    base = wid * _PW + chunk * _C
    pltpu.sync_copy(xt_hbm.at[0, pl.ds(base, _C)], xv)
    pltpu.sync_copy(xt_hbm.at[1, pl.ds(base, _C)], yv)
    for g in range(_G):
        sl = pl.ds(g * _L, _L)
        xs = xv[sl]
        ys = yv[sl]
        x0 = xs.astype(jnp.int32)
        y0 = ys.astype(jnp.int32)
        tx = xs - x0.astype(jnp.float32)
        ty = ys - y0.astype(jnp.float32)
        c00 = x0 * _H + y0
        idxb[pl.ds(0 * _C + g * _L, _L)] = c00
        idxb[pl.ds(1 * _C + g * _L, _L)] = c00 + _H
        wb[pl.ds(0 * _C + g * _L, _L)] = tx
        wb[pl.ds(1 * _C + g * _L, _L)] = ty
    for c in range(2):
        pltpu.async_copy(dup_hbm.at[idxb.at[pl.ds(c * _C, _C)]], rows.at[c],
                         gsem)


def _compute(half, dup_hbm, idxb, wb, rows, ob, gsem):
    for c in range(2):
        pltpu.make_async_copy(dup_hbm.at[idxb.at[pl.ds(c * _C, _C)]],
                              rows.at[c], gsem).wait()

    lane = lax.iota(jnp.int32, _L)
    zeros = jnp.zeros((_L,), jnp.int32)

    @plsc.parallel_loop(0, _C, unroll=2)
    def body(i):
        iv = zeros + i
        txv = plsc.load_gather(wb, [iv])
        tyv = plsc.load_gather(wb, [iv + _C])
        col = zeros + (i + half * _C)
        for k in range(_D // _L):
            lo = pl.ds(k * _L, _L)
            hi = pl.ds(_D + k * _L, _L)
            p00 = rows[0, i, lo]
            p01 = rows[0, i, hi]
            p10 = rows[1, i, lo]
            p11 = rows[1, i, hi]
            top = p00 + txv * (p10 - p00)
            bot = p01 + txv * (p11 - p01)
            res = top + tyv * (bot - top)
            plsc.store_scatter(ob, [lane + k * _L, col], res)


@functools.partial(
    pl.kernel,
    out_type=jax.ShapeDtypeStruct((_D, _N), jnp.float32),
    mesh=_mesh,
    compiler_params=_params,
    scratch_types=[
        pltpu.VMEM((_C,), jnp.float32),
        pltpu.VMEM((_C,), jnp.float32),
        pltpu.VMEM((2 * _C,), jnp.int32),
        pltpu.VMEM((2 * _C,), jnp.float32),
        pltpu.VMEM((2, _C, 2 * _D), jnp.float32),
        pltpu.SemaphoreType.DMA,
        pltpu.VMEM((_C,), jnp.float32),
        pltpu.VMEM((_C,), jnp.float32),
        pltpu.VMEM((2 * _C,), jnp.int32),
        pltpu.VMEM((2 * _C,), jnp.float32),
        pltpu.VMEM((2, _C, 2 * _D), jnp.float32),
        pltpu.SemaphoreType.DMA,
        pltpu.VMEM((2, _D, 2 * _C), jnp.float32),
        pltpu.SemaphoreType.DMA,
        pltpu.SemaphoreType.DMA,
    ],
)
def _bilerp_sc(xt_hbm, dup_hbm, out_hbm,
               xvA, yvA, idxA, wbA, rowsA, gsemA,
               xvB, yvB, idxB, wbB, rowsB, gsemB, ob, osem0, osem1):
    wid = lax.axis_index("s") * _NC + lax.axis_index("c")
    slotA = (xvA, yvA, idxA, wbA, rowsA, gsemA)
    slotB = (xvB, yvB, idxB, wbB, rowsB, gsemB)
    _prepare(0, wid, xt_hbm, dup_hbm, *slotA)

    def quad(q, carry):
        for s, osem in ((0, osem0), (1, osem1)):
            g = q * 4 + s * 2
            obs = ob.at[s]
            colbase = wid * _PW + g * _C
            dst = out_hbm.at[:, pl.ds(colbase, 2 * _C)]
            _prepare(g + 1, wid, xt_hbm, dup_hbm, *slotB)

            @pl.when(q > 0)
            def _():
                pltpu.make_async_copy(obs, dst, osem).wait()

            _compute(0, dup_hbm, idxA, wbA, rowsA, obs, gsemA)

            @pl.when(g + 2 < _NCH)
            def _():
                _prepare(g + 2, wid, xt_hbm, dup_hbm, *slotA)

            _compute(1, dup_hbm, idxB, wbB, rowsB, obs, gsemB)
            pltpu.async_copy(obs, dst, osem)
        return carry

    lax.fori_loop(0, _NCH // 4, quad, 0)
    final = out_hbm.at[:, pl.ds(wid * _PW, 2 * _C)]
    pltpu.make_async_copy(ob.at[0], final, osem0).wait()
    pltpu.make_async_copy(ob.at[1], final, osem1).wait()


def kernel(x, plane):
    dup = _build_pairs(plane)
    out_t = _bilerp_sc(x.T, dup)
    return out_t.T


# speedup vs baseline: 1.8032x; 1.0052x over previous
import functools

import jax
import jax.numpy as jnp
from jax import lax
from jax.experimental import pallas as pl
from jax.experimental.pallas import tpu as pltpu
from jax.experimental.pallas import tpu_sc as plsc

_W, _H, _D = 1024, 1024, 64
_N = 524288
_NC = 2
_NS = 16
_NW = _NC * _NS
_L = 16
_CELLS = _W * _H

_RB = 128
_CELLW = _CELLS // _NW
_NBCH = _CELLW // _RB

_C = 128
_PW = _N // _NW
_NCH = _PW // _C
_G = _C // _L

_params = pltpu.CompilerParams(
    needs_layout_passes=False, use_tc_tiling_on_sc=True)
_mesh = plsc.VectorSubcoreMesh(core_axis_name="c", subcore_axis_name="s")


def _bp_prepare(chunk, wid, p64, stage, sem):
    base = wid * _CELLW + chunk * _RB
    pltpu.async_copy(p64.at[pl.ds(base, _RB)], stage.at[pl.ds(0, _RB)], sem)

    @pl.when(base + _RB < _CELLS)
    def _():
        pltpu.async_copy(p64.at[pl.ds(base + _RB, 8)],
                         stage.at[pl.ds(_RB, 8)], sem)


def _bp_compute(chunk, wid, p64, dup_hbm, stage, ob, sem):
    base = wid * _CELLW + chunk * _RB
    pltpu.make_async_copy(p64.at[pl.ds(base, _RB)], stage.at[pl.ds(0, _RB)],
                          sem).wait()

    @pl.when(base + _RB < _CELLS)
    def _():
        pltpu.make_async_copy(p64.at[pl.ds(base + _RB, 8)],
                              stage.at[pl.ds(_RB, 8)], sem).wait()

    @plsc.parallel_loop(0, _RB, unroll=2)
    def body(r):
        for k in range(_D // _L):
            sl = pl.ds(k * _L, _L)
            ob[r, sl] = stage[r, sl]
        for k in range(_D // _L):
            sl = pl.ds(k * _L, _L)
            ob[r, pl.ds(_D + k * _L, _L)] = stage[r + 1, sl]

    pltpu.sync_copy(ob, dup_hbm.at[pl.ds(base, _RB)])


@functools.partial(
    pl.kernel,
    out_type=jax.ShapeDtypeStruct((_CELLS, 2 * _D), jnp.float32),
    mesh=_mesh,
    compiler_params=_params,
    scratch_types=[
        pltpu.VMEM((_RB + 8, _D), jnp.float32),
        pltpu.VMEM((_RB, 2 * _D), jnp.float32),
        pltpu.SemaphoreType.DMA,
        pltpu.VMEM((_RB + 8, _D), jnp.float32),
        pltpu.VMEM((_RB, 2 * _D), jnp.float32),
        pltpu.SemaphoreType.DMA,
    ],
)
def _build_pairs(plane3d, dup_hbm,
                 stageA, obA, semA, stageB, obB, semB):
    p64 = plane3d.reshape(_CELLS, _D)
    wid = lax.axis_index("s") * _NC + lax.axis_index("c")
    _bp_prepare(0, wid, p64, stageA, semA)

    def pair(p, carry):
        g = p * 2
        _bp_prepare(g + 1, wid, p64, stageB, semB)
        _bp_compute(g, wid, p64, dup_hbm, stageA, obA, semA)

        @pl.when(g + 2 < _NBCH)
        def _():
            _bp_prepare(g + 2, wid, p64, stageA, semA)

        _bp_compute(g + 1, wid, p64, dup_hbm, stageB, obB, semB)
        return carry

    lax.fori_loop(0, _NBCH // 2, pair, 0)


def _prepare(chunk, wid, xt_hbm, dup_hbm, xv, yv, idxb, wb, rows, gsem):
    base = wid * _PW + chunk * _C
    pltpu.sync_copy(xt_hbm.at[0, pl.ds(base, _C)], xv)
    pltpu.sync_copy(xt_hbm.at[1, pl.ds(base, _C)], yv)
    for g in range(_G):
        sl = pl.ds(g * _L, _L)
        xs = xv[sl]
        ys = yv[sl]
        x0 = xs.astype(jnp.int32)
        y0 = ys.astype(jnp.int32)
        tx = xs - x0.astype(jnp.float32)
        ty = ys - y0.astype(jnp.float32)
        c00 = x0 * _H + y0
        idxb[pl.ds(0 * _C + g * _L, _L)] = c00
        idxb[pl.ds(1 * _C + g * _L, _L)] = c00 + _H
        wb[pl.ds(0 * _C + g * _L, _L)] = tx
        wb[pl.ds(1 * _C + g * _L, _L)] = ty
    for c in range(2):
        pltpu.async_copy(dup_hbm.at[idxb.at[pl.ds(c * _C, _C)]], rows.at[c],
                         gsem)


def _compute(half, dup_hbm, idxb, wb, rows, ob, gsem):
    for c in range(2):
        pltpu.make_async_copy(dup_hbm.at[idxb.at[pl.ds(c * _C, _C)]],
                              rows.at[c], gsem).wait()

    lane = lax.iota(jnp.int32, _L)
    zeros = jnp.zeros((_L,), jnp.int32)

    @plsc.parallel_loop(0, _C, unroll=4)
    def body(i):
        iv = zeros + i
        txv = plsc.load_gather(wb, [iv])
        tyv = plsc.load_gather(wb, [iv + _C])
        col = zeros + (i + half * _C)
        for k in range(_D // _L):
            lo = pl.ds(k * _L, _L)
            hi = pl.ds(_D + k * _L, _L)
            p00 = rows[0, i, lo]
            p01 = rows[0, i, hi]
            p10 = rows[1, i, lo]
            p11 = rows[1, i, hi]
            top = p00 + txv * (p10 - p00)
            bot = p01 + txv * (p11 - p01)
            res = top + tyv * (bot - top)
            plsc.store_scatter(ob, [lane + k * _L, col], res)


@functools.partial(
    pl.kernel,
    out_type=jax.ShapeDtypeStruct((_D, _N), jnp.float32),
    mesh=_mesh,
    compiler_params=_params,
    scratch_types=[
        pltpu.VMEM((_C,), jnp.float32),
        pltpu.VMEM((_C,), jnp.float32),
        pltpu.VMEM((2 * _C,), jnp.int32),
        pltpu.VMEM((2 * _C,), jnp.float32),
        pltpu.VMEM((2, _C, 2 * _D), jnp.float32),
        pltpu.SemaphoreType.DMA,
        pltpu.VMEM((_C,), jnp.float32),
        pltpu.VMEM((_C,), jnp.float32),
        pltpu.VMEM((2 * _C,), jnp.int32),
        pltpu.VMEM((2 * _C,), jnp.float32),
        pltpu.VMEM((2, _C, 2 * _D), jnp.float32),
        pltpu.SemaphoreType.DMA,
        pltpu.VMEM((2, _D, 2 * _C), jnp.float32),
        pltpu.SemaphoreType.DMA,
        pltpu.SemaphoreType.DMA,
    ],
)
def _bilerp_sc(xt_hbm, dup_hbm, out_hbm,
               xvA, yvA, idxA, wbA, rowsA, gsemA,
               xvB, yvB, idxB, wbB, rowsB, gsemB, ob, osem0, osem1):
    wid = lax.axis_index("s") * _NC + lax.axis_index("c")
    slotA = (xvA, yvA, idxA, wbA, rowsA, gsemA)
    slotB = (xvB, yvB, idxB, wbB, rowsB, gsemB)
    _prepare(0, wid, xt_hbm, dup_hbm, *slotA)

    def quad(q, carry):
        for s, osem in ((0, osem0), (1, osem1)):
            g = q * 4 + s * 2
            obs = ob.at[s]
            colbase = wid * _PW + g * _C
            dst = out_hbm.at[:, pl.ds(colbase, 2 * _C)]
            _prepare(g + 1, wid, xt_hbm, dup_hbm, *slotB)

            @pl.when(q > 0)
            def _():
                pltpu.make_async_copy(obs, dst, osem).wait()

            _compute(0, dup_hbm, idxA, wbA, rowsA, obs, gsemA)

            @pl.when(g + 2 < _NCH)
            def _():
                _prepare(g + 2, wid, xt_hbm, dup_hbm, *slotA)

            _compute(1, dup_hbm, idxB, wbB, rowsB, obs, gsemB)
            pltpu.async_copy(obs, dst, osem)
        return carry

    lax.fori_loop(0, _NCH // 4, quad, 0)
    final = out_hbm.at[:, pl.ds(wid * _PW, 2 * _C)]
    pltpu.make_async_copy(ob.at[0], final, osem0).wait()
    pltpu.make_async_copy(ob.at[1], final, osem1).wait()


def kernel(x, plane):
    dup = _build_pairs(plane)
    out_t = _bilerp_sc(x.T, dup)
    return out_t.T
